# baseline (speedup 1.0000x reference)
import jax
import jax.numpy as jnp
from jax import lax
from jax.experimental import pallas as pl
from jax.experimental.pallas import tpu as pltpu
from jax.experimental.pallas import tpu_sc as plsc

_N = 320000
_D = 128
_MAX_LEN = 240

_NUM_WORKERS = 32
_ROWS_PER_WORKER = _N // _NUM_WORKERS
_C = 80
_CHUNK = _C * _D
_NITER = _ROWS_PER_WORKER // _C
_NBUF = 5
_PF = 3


def _fuse_table_kernel(emb_ref, w_ref, b_ref, out_ref):
    out_ref[:, :] = (
        lax.dot_general(
            emb_ref[:, :], w_ref[:, :],
            dimension_numbers=(((1,), (1,)), ((), ())),
            preferred_element_type=jnp.float32,
        )
        + b_ref[:, :]
    )


def _sc_body(x_hbm, t_hbm, tab_hbm, out_hbm, idx_v, *bufs):
    xb = bufs[0:_NBUF]
    gb = bufs[_NBUF:2 * _NBUF]
    x_sem = bufs[2 * _NBUF:3 * _NBUF]
    g_sem = bufs[3 * _NBUF:4 * _NBUF]
    o_sem = bufs[4 * _NBUF:5 * _NBUF]

    wid = lax.axis_index("s") * 2 + lax.axis_index("c")
    row_base = wid * _ROWS_PER_WORKER

    pltpu.sync_copy(t_hbm.at[pl.ds(row_base, _ROWS_PER_WORKER)], idx_v)

    def x_slice(c):
        return x_hbm.at[pl.ds(row_base + c * _C, _C), :]

    def out_slice(c):
        return out_hbm.at[pl.ds(row_base + c * _C, _C), :]

    def start_chunk(c, b):
        pltpu.async_copy(x_slice(c), xb[b], x_sem[b])
        pltpu.async_copy(
            tab_hbm.at[idx_v.at[pl.ds(c * _C, _C)]], gb[b], g_sem[b]
        )

    for i in range(_PF):
        start_chunk(i, i)

    def outer(k, carry):
        for bi in range(_NBUF):
            c = k * _NBUF + bi
            pf = c + _PF
            bpf = (bi + _PF) % _NBUF

            @pl.when(pf < _NITER)
            def _prefetch():
                pltpu.async_copy(x_slice(pf), xb[bpf], x_sem[bpf])

                @pl.when(pf >= _NBUF)
                def _drain():
                    pltpu.make_async_copy(
                        gb[bpf], out_slice(pf - _NBUF), o_sem[bpf]
                    ).wait()

                pltpu.async_copy(
                    tab_hbm.at[idx_v.at[pl.ds(pf * _C, _C)]],
                    gb[bpf], g_sem[bpf],
                )

            pltpu.make_async_copy(x_slice(c), xb[bi], x_sem[bi]).wait()
            pltpu.make_async_copy(
                tab_hbm.at[idx_v.at[pl.ds(c * _C, _C)]], gb[bi], g_sem[bi]
            ).wait()

            @plsc.parallel_loop(0, _C, 1, unroll=2)
            def _add(r, xv=xb[bi], gv=gb[bi]):
                for j in range(_D // 16):
                    sl = pl.ds(j * 16, 16)
                    plsc.addupdate(gv.at[r, sl], xv[r, sl])

            pltpu.async_copy(gb[bi], out_slice(c), o_sem[bi])
        return carry

    lax.fori_loop(0, _NITER // _NBUF, outer, 0, unroll=False)

    for bi in range(_NBUF):
        c_last = _NITER - _NBUF + bi
        pltpu.make_async_copy(gb[bi], out_slice(c_last), o_sem[bi]).wait()


def kernel(x, t, emb_table, W, b):
    fused_table = pl.pallas_call(
        _fuse_table_kernel,
        out_shape=jax.ShapeDtypeStruct((_MAX_LEN, _D), jnp.float32),
    )(emb_table, W, b.reshape(1, _D))

    mesh = plsc.VectorSubcoreMesh(core_axis_name="c", subcore_axis_name="s")
    scratch = (
        [pltpu.VMEM((_ROWS_PER_WORKER,), jnp.int32)]
        + [pltpu.VMEM((_C, _D), jnp.float32) for _ in range(2 * _NBUF)]
        + [pltpu.SemaphoreType.DMA for _ in range(3 * _NBUF)]
    )
    sc_gather_add = pl.kernel(
        _sc_body,
        out_type=jax.ShapeDtypeStruct((_N, _D), jnp.float32),
        mesh=mesh,
        scratch_types=scratch,
        compiler_params=pltpu.CompilerParams(needs_layout_passes=False),
    )
    return sc_gather_add(x, t, fused_table)


# speedup vs baseline: 2.9571x; 2.9571x over previous
import jax
import jax.numpy as jnp
from jax import lax
from jax.experimental import pallas as pl
from jax.experimental.pallas import tpu as pltpu
from jax.experimental.pallas import tpu_sc as plsc

_N = 320000
_D = 128
_MAX_LEN = 240

_NUM_WORKERS = 32
_ROWS_PER_WORKER = _N // _NUM_WORKERS
_C = 80
_CHUNK = _C * _D
_NITER = _ROWS_PER_WORKER // _C
_NBUF = 5
_PF = 3


def _fuse_table_kernel(emb_ref, w_ref, b_ref, out_ref):
    out_ref[:, :] = (
        lax.dot_general(
            emb_ref[:, :], w_ref[:, :],
            dimension_numbers=(((1,), (1,)), ((), ())),
            preferred_element_type=jnp.float32,
        )
        + b_ref[:, :]
    )


def _sc_body(x_hbm, t_hbm, tab_hbm, out_hbm, tab_sh, idx_v, *bufs):
    xb = bufs[0:_NBUF]
    gb = bufs[_NBUF:2 * _NBUF]
    x_sem = bufs[2 * _NBUF:3 * _NBUF]
    g_sem = bufs[3 * _NBUF:4 * _NBUF]
    o_sem = bufs[4 * _NBUF:5 * _NBUF]

    wid = lax.axis_index("s") * 2 + lax.axis_index("c")
    row_base = wid * _ROWS_PER_WORKER

    pltpu.sync_copy(t_hbm.at[pl.ds(row_base, _ROWS_PER_WORKER)], idx_v)

    @pl.when(lax.axis_index("s") == 0)
    def _stage_table():
        pltpu.sync_copy(tab_hbm, tab_sh)

    plsc.subcore_barrier()

    def x_slice(c):
        return x_hbm.at[pl.ds(row_base + c * _C, _C), :]

    def out_slice(c):
        return out_hbm.at[pl.ds(row_base + c * _C, _C), :]

    def start_chunk(c, b):
        pltpu.async_copy(x_slice(c), xb[b], x_sem[b])
        pltpu.async_copy(
            tab_sh.at[idx_v.at[pl.ds(c * _C, _C)]], gb[b], g_sem[b]
        )

    for i in range(_PF):
        start_chunk(i, i)

    def outer(k, carry):
        for bi in range(_NBUF):
            c = k * _NBUF + bi
            pf = c + _PF
            bpf = (bi + _PF) % _NBUF

            @pl.when(pf < _NITER)
            def _prefetch():
                pltpu.async_copy(x_slice(pf), xb[bpf], x_sem[bpf])

                @pl.when(pf >= _NBUF)
                def _drain():
                    pltpu.make_async_copy(
                        gb[bpf], out_slice(pf - _NBUF), o_sem[bpf]
                    ).wait()

                pltpu.async_copy(
                    tab_sh.at[idx_v.at[pl.ds(pf * _C, _C)]],
                    gb[bpf], g_sem[bpf],
                )

            pltpu.make_async_copy(x_slice(c), xb[bi], x_sem[bi]).wait()
            pltpu.make_async_copy(
                tab_sh.at[idx_v.at[pl.ds(c * _C, _C)]], gb[bi], g_sem[bi]
            ).wait()

            @plsc.parallel_loop(0, _C, 1, unroll=2)
            def _add(r, xv=xb[bi], gv=gb[bi]):
                for j in range(_D // 16):
                    sl = pl.ds(j * 16, 16)
                    plsc.addupdate(gv.at[r, sl], xv[r, sl])

            pltpu.async_copy(gb[bi], out_slice(c), o_sem[bi])
        return carry

    lax.fori_loop(0, _NITER // _NBUF, outer, 0, unroll=False)

    for bi in range(_NBUF):
        c_last = _NITER - _NBUF + bi
        pltpu.make_async_copy(gb[bi], out_slice(c_last), o_sem[bi]).wait()


def kernel(x, t, emb_table, W, b):
    fused_table = pl.pallas_call(
        _fuse_table_kernel,
        out_shape=jax.ShapeDtypeStruct((_MAX_LEN, _D), jnp.float32),
    )(emb_table, W, b.reshape(1, _D))

    mesh = plsc.VectorSubcoreMesh(core_axis_name="c", subcore_axis_name="s")
    scratch = (
        [pltpu.VMEM_SHARED((_MAX_LEN, _D), jnp.float32)]
        + [pltpu.VMEM((_ROWS_PER_WORKER,), jnp.int32)]
        + [pltpu.VMEM((_C, _D), jnp.float32) for _ in range(2 * _NBUF)]
        + [pltpu.SemaphoreType.DMA for _ in range(3 * _NBUF)]
    )
    sc_gather_add = pl.kernel(
        _sc_body,
        out_type=jax.ShapeDtypeStruct((_N, _D), jnp.float32),
        mesh=mesh,
        scratch_types=scratch,
        compiler_params=pltpu.CompilerParams(needs_layout_passes=False),
    )
    return sc_gather_add(x, t, fused_table)


# speedup vs baseline: 2.9876x; 1.0103x over previous
import jax
import jax.numpy as jnp
from jax import lax
from jax.experimental import pallas as pl
from jax.experimental.pallas import tpu as pltpu
from jax.experimental.pallas import tpu_sc as plsc

_N = 320000
_D = 128
_MAX_LEN = 240

_NUM_WORKERS = 32
_ROWS_PER_WORKER = _N // _NUM_WORKERS
_C = 80
_CHUNK = _C * _D
_NITER = _ROWS_PER_WORKER // _C
_NBUF = 5
_PF = 3


def _fuse_table_kernel(emb_ref, w_ref, b_ref, out_ref):
    out_ref[:, :] = (
        lax.dot_general(
            emb_ref[:, :], w_ref[:, :],
            dimension_numbers=(((1,), (1,)), ((), ())),
            preferred_element_type=jnp.float32,
        )
        + b_ref[:, :]
    )


def _sc_body(x_hbm, t_hbm, tab_hbm, out_hbm, tab_sh, idx_v, *bufs):
    xb = bufs[0:_NBUF]
    x_sem = bufs[_NBUF:2 * _NBUF]
    g_sem = bufs[2 * _NBUF:3 * _NBUF]
    o_sem = bufs[3 * _NBUF:4 * _NBUF]

    wid = lax.axis_index("s") * 2 + lax.axis_index("c")
    row_base = wid * _ROWS_PER_WORKER

    pltpu.sync_copy(t_hbm.at[pl.ds(row_base, _ROWS_PER_WORKER)], idx_v)

    @pl.when(lax.axis_index("s") == 0)
    def _stage_table():
        pltpu.sync_copy(tab_hbm, tab_sh)

    plsc.subcore_barrier()

    def x_slice(c):
        return x_hbm.at[pl.ds(row_base + c * _C, _C), :]

    def out_slice(c):
        return out_hbm.at[pl.ds(row_base + c * _C, _C), :]

    def gadd(c, b):
        pltpu.async_copy(
            tab_sh.at[idx_v.at[pl.ds(c * _C, _C)]], xb[b], g_sem[b],
            add=True,
        )

    for i in range(_PF):
        pltpu.async_copy(x_slice(i), xb[i], x_sem[i])
    pltpu.make_async_copy(x_slice(0), xb[0], x_sem[0]).wait()
    gadd(0, 0)

    def outer(k, carry):
        for bi in range(_NBUF):
            c = k * _NBUF + bi
            bn = (bi + 1) % _NBUF
            pf = c + _PF
            bpf = (bi + _PF) % _NBUF

            @pl.when(pf < _NITER)
            def _prefetch():
                @pl.when(pf >= _NBUF)
                def _drain():
                    pltpu.make_async_copy(
                        xb[bpf], out_slice(pf - _NBUF), o_sem[bpf]
                    ).wait()

                pltpu.async_copy(x_slice(pf), xb[bpf], x_sem[bpf])

            @pl.when(c + 1 < _NITER)
            def _next_gadd():
                pltpu.make_async_copy(
                    x_slice(c + 1), xb[bn], x_sem[bn]
                ).wait()
                gadd(c + 1, bn)

            pltpu.make_async_copy(
                tab_sh.at[idx_v.at[pl.ds(c * _C, _C)]], xb[bi], g_sem[bi]
            ).wait()
            pltpu.async_copy(xb[bi], out_slice(c), o_sem[bi])
        return carry

    lax.fori_loop(0, _NITER // _NBUF, outer, 0, unroll=False)

    for bi in range(_NBUF):
        c_last = _NITER - _NBUF + bi
        pltpu.make_async_copy(xb[bi], out_slice(c_last), o_sem[bi]).wait()


def kernel(x, t, emb_table, W, b):
    fused_table = pl.pallas_call(
        _fuse_table_kernel,
        out_shape=jax.ShapeDtypeStruct((_MAX_LEN, _D), jnp.float32),
    )(emb_table, W, b.reshape(1, _D))

    mesh = plsc.VectorSubcoreMesh(core_axis_name="c", subcore_axis_name="s")
    scratch = (
        [pltpu.VMEM_SHARED((_MAX_LEN, _D), jnp.float32)]
        + [pltpu.VMEM((_ROWS_PER_WORKER,), jnp.int32)]
        + [pltpu.VMEM((_C, _D), jnp.float32) for _ in range(_NBUF)]
        + [pltpu.SemaphoreType.DMA for _ in range(3 * _NBUF)]
    )
    sc_gather_add = pl.kernel(
        _sc_body,
        out_type=jax.ShapeDtypeStruct((_N, _D), jnp.float32),
        mesh=mesh,
        scratch_types=scratch,
        compiler_params=pltpu.CompilerParams(needs_layout_passes=False),
    )
    return sc_gather_add(x, t, fused_table)
